# baseline (speedup 1.0000x reference)
import functools

import jax
import jax.numpy as jnp
from jax import lax
from jax.experimental import pallas as pl
from jax.experimental.pallas import tpu as pltpu
from jax.experimental.pallas import tpu_sc as plsc

_LANES = 16
_NCORES = 2
_NSUB = 16
_NW = _NCORES * _NSUB
_NBUF = 4
_RED_UNROLL = 8


@functools.lru_cache(maxsize=None)
def _build(B, Lp, V, E):
    bpw = B // _NW
    nchunk = 2
    chunk = Lp // nchunk
    ek = E // _LANES
    assert B % _NW == 0 and Lp % _LANES == 0 and chunk % 8 == 0 and chunk <= 128
    assert E % _LANES == 0 and bpw % _NBUF == 0 and Lp % _RED_UNROLL == 0

    mesh = plsc.VectorSubcoreMesh(core_axis_name="c", subcore_axis_name="s")

    @functools.partial(
        pl.kernel,
        mesh=mesh,
        compiler_params=pltpu.CompilerParams(
            use_tc_tiling_on_sc=False, needs_layout_passes=False
        ),
        out_type=jax.ShapeDtypeStruct((B, E), jnp.float32),
        scratch_types=[
            pltpu.VMEM((bpw, Lp), jnp.int32),
            pltpu.VMEM((_NBUF, Lp, E), jnp.float32),
            pltpu.VMEM((bpw, E), jnp.float32),
        ] + [pltpu.SemaphoreType.DMA] * _NBUF,
    )
    def enc(x_hbm, table_hbm, out_hbm, x_v, rows_v, out_v, *sems):
        wid = lax.axis_index("s") * _NCORES + lax.axis_index("c")
        base = wid * bpw
        pltpu.sync_copy(x_hbm.at[pl.ds(base, bpw)], x_v)

        def start_gather(b, buf):
            for c in range(nchunk):
                pltpu.async_copy(
                    table_hbm.at[x_v.at[b, pl.ds(c * chunk, chunk)]],
                    rows_v.at[buf, pl.ds(c * chunk, chunk)],
                    sems[buf],
                )

        def wait_gather(buf):
            for c in range(nchunk):
                pltpu.make_async_copy(
                    table_hbm.at[x_v.at[0, pl.ds(c * chunk, chunk)]],
                    rows_v.at[buf, pl.ds(c * chunk, chunk)],
                    sems[buf],
                ).wait()

        def process(b, buf):
            def red_body(i, accs):
                accs = list(accs)
                for u in range(_RED_UNROLL):
                    r = i * _RED_UNROLL + u
                    for k in range(ek):
                        accs[k] = accs[k] + rows_v[buf, r, pl.ds(k * _LANES, _LANES)]
                return tuple(accs)

            zeros = jnp.zeros((_LANES,), jnp.float32)
            accs = lax.fori_loop(0, Lp // _RED_UNROLL, red_body, (zeros,) * ek)

            cnt = jnp.zeros((_LANES,), jnp.int32)
            for j in range(Lp // _LANES):
                v = x_v[b, pl.ds(j * _LANES, _LANES)]
                cnt = cnt + plsc.all_reduce_population_count(v != 0)
            inv = 1.0 / jnp.maximum(cnt.astype(jnp.float32), 1.0)
            for k in range(ek):
                out_v[b, pl.ds(k * _LANES, _LANES)] = accs[k] * inv

        for j in range(_NBUF - 1):
            start_gather(j, j)

        def outer(g, carry):
            for u in range(_NBUF):
                b = g * _NBUF + u
                wait_gather(u)
                nb = b + (_NBUF - 1)

                @pl.when(nb < bpw)
                def _():
                    start_gather(nb, (u + _NBUF - 1) % _NBUF)

                process(b, u)
            return carry

        lax.fori_loop(0, bpw // _NBUF, outer, 0)
        pltpu.sync_copy(out_v, out_hbm.at[pl.ds(base, bpw)])

    return enc


def kernel(x, lengths, table):
    del lengths
    B, L = x.shape
    V, E = table.shape
    Lp = ((L + _LANES - 1) // _LANES) * _LANES
    xp = jnp.pad(x.astype(jnp.int32), ((0, 0), (0, Lp - L)))
    return _build(B, Lp, V, E)(xp, table)


# speedup vs baseline: 1.9601x; 1.9601x over previous
import functools

import jax
import jax.numpy as jnp
from jax import lax
from jax.experimental import pallas as pl
from jax.experimental.pallas import tpu as pltpu
from jax.experimental.pallas import tpu_sc as plsc

_LANES = 16
_NCORES = 2
_NSUB = 16
_NW = _NCORES * _NSUB
_NBUF = 4
_RED_UNROLL = 8


def _chunks(L):
    out, off = [], 0
    while off < L:
        size = min(128, L - off)
        if off + size < L:
            size -= size % 8
        out.append((off, size))
        off += size
    return tuple(out)


@functools.lru_cache(maxsize=None)
def _build(B, L, V, E):
    bpw = B // _NW
    ek = E // _LANES
    chunks = _chunks(L)
    assert B % _NW == 0 and E % _LANES == 0 and bpw % _NBUF == 0
    assert all(off % 8 == 0 and sz <= 128 for off, sz in chunks)

    nfull = L // _LANES
    tail = L - nfull * _LANES

    mesh = plsc.VectorSubcoreMesh(core_axis_name="c", subcore_axis_name="s")

    @functools.partial(
        pl.kernel,
        mesh=mesh,
        compiler_params=pltpu.CompilerParams(
            use_tc_tiling_on_sc=False, needs_layout_passes=False
        ),
        out_type=jax.ShapeDtypeStruct((B, E), jnp.float32),
        scratch_types=[
            pltpu.VMEM((bpw, L), jnp.int32),
            pltpu.VMEM((_NBUF, L, E), jnp.float32),
            pltpu.VMEM((bpw, E), jnp.float32),
        ] + [pltpu.SemaphoreType.DMA] * _NBUF,
    )
    def enc(x_hbm, table_hbm, out_hbm, x_v, rows_v, out_v, *sems):
        wid = lax.axis_index("s") * _NCORES + lax.axis_index("c")
        base = wid * bpw
        pltpu.sync_copy(x_hbm.at[pl.ds(base, bpw)], x_v)

        def start_gather(b, buf):
            for off, sz in chunks:
                pltpu.async_copy(
                    table_hbm.at[x_v.at[b, pl.ds(off, sz)]],
                    rows_v.at[buf, pl.ds(off, sz)],
                    sems[buf],
                )

        def wait_gather(buf):
            for off, sz in chunks:
                pltpu.make_async_copy(
                    table_hbm.at[x_v.at[0, pl.ds(off, sz)]],
                    rows_v.at[buf, pl.ds(off, sz)],
                    sems[buf],
                ).wait()

        def process(b, buf):
            def red_body(i, accs):
                accs = list(accs)
                for u in range(_RED_UNROLL):
                    r = i * _RED_UNROLL + u
                    for k in range(ek):
                        accs[k] = accs[k] + rows_v[buf, r, pl.ds(k * _LANES, _LANES)]
                return tuple(accs)

            zeros = jnp.zeros((_LANES,), jnp.float32)
            accs = lax.fori_loop(0, L // _RED_UNROLL, red_body, (zeros,) * ek)
            for r in range((L // _RED_UNROLL) * _RED_UNROLL, L):
                accs = tuple(
                    accs[k] + rows_v[buf, r, pl.ds(k * _LANES, _LANES)]
                    for k in range(ek)
                )

            cnt = jnp.zeros((_LANES,), jnp.int32)
            for j in range(nfull):
                v = x_v[b, pl.ds(j * _LANES, _LANES)]
                cnt = cnt + plsc.all_reduce_population_count(v != 0)
            if tail:
                v = x_v[b, pl.ds(L - _LANES, _LANES)]
                m = (v != 0) & (lax.iota(jnp.int32, _LANES) >= (_LANES - tail))
                cnt = cnt + plsc.all_reduce_population_count(m)
            inv = 1.0 / jnp.maximum(cnt.astype(jnp.float32), 1.0)
            for k in range(ek):
                out_v[b, pl.ds(k * _LANES, _LANES)] = accs[k] * inv

        for j in range(_NBUF - 1):
            start_gather(j, j)

        def outer(g, carry):
            for u in range(_NBUF):
                b = g * _NBUF + u
                wait_gather(u)
                nb = b + (_NBUF - 1)

                @pl.when(nb < bpw)
                def _():
                    start_gather(nb, (u + _NBUF - 1) % _NBUF)

                process(b, u)
            return carry

        lax.fori_loop(0, bpw // _NBUF, outer, 0)
        pltpu.sync_copy(out_v, out_hbm.at[pl.ds(base, bpw)])

    return enc


def kernel(x, lengths, table):
    del lengths
    B, L = x.shape
    V, E = table.shape
    return _build(B, L, V, E)(x.astype(jnp.int32), table)


# speedup vs baseline: 1.9616x; 1.0008x over previous
import functools

import jax
import jax.numpy as jnp
from jax import lax
from jax.experimental import pallas as pl
from jax.experimental.pallas import tpu as pltpu
from jax.experimental.pallas import tpu_sc as plsc

_LANES = 16
_NCORES = 2
_NSUB = 16
_NW = _NCORES * _NSUB
_NBUF = 8
_RED_UNROLL = 8


def _chunks(L):
    out, off = [], 0
    while off < L:
        size = min(128, L - off)
        if off + size < L:
            size -= size % 8
        out.append((off, size))
        off += size
    return tuple(out)


@functools.lru_cache(maxsize=None)
def _build(B, L, V, E):
    bpw = B // _NW
    ek = E // _LANES
    chunks = _chunks(L)
    nck = len(chunks)
    rows_per_group = _NBUF // nck
    assert B % _NW == 0 and E % _LANES == 0 and _NBUF % nck == 0
    assert bpw % rows_per_group == 0
    assert all(off % 8 == 0 and sz <= 128 for off, sz in chunks)

    nfull = L // _LANES
    tail = L - nfull * _LANES

    mesh = plsc.VectorSubcoreMesh(core_axis_name="c", subcore_axis_name="s")

    @functools.partial(
        pl.kernel,
        mesh=mesh,
        compiler_params=pltpu.CompilerParams(
            use_tc_tiling_on_sc=False, needs_layout_passes=False
        ),
        out_type=jax.ShapeDtypeStruct((B, E), jnp.float32),
        scratch_types=[
            pltpu.VMEM((bpw, L), jnp.int32),
            pltpu.VMEM((_NBUF, 128, E), jnp.float32),
            pltpu.VMEM((bpw, E), jnp.float32),
        ] + [pltpu.SemaphoreType.DMA] * _NBUF,
    )
    def enc(x_hbm, table_hbm, out_hbm, x_v, rows_v, out_v, *sems):
        wid = lax.axis_index("s") * _NCORES + lax.axis_index("c")
        base = wid * bpw
        pltpu.sync_copy(x_hbm.at[pl.ds(base, bpw)], x_v)

        def start_gather(b, ci, buf):
            off, sz = chunks[ci]
            pltpu.async_copy(
                table_hbm.at[x_v.at[b, pl.ds(off, sz)]],
                rows_v.at[buf, pl.ds(0, sz)],
                sems[buf],
            )

        def wait_gather(ci, buf):
            off, sz = chunks[ci]
            pltpu.make_async_copy(
                table_hbm.at[x_v.at[0, pl.ds(off, sz)]],
                rows_v.at[buf, pl.ds(0, sz)],
                sems[buf],
            ).wait()

        def accumulate(buf, sz, accs):
            def red_body(i, accs):
                accs = list(accs)
                for u in range(_RED_UNROLL):
                    r = i * _RED_UNROLL + u
                    for k in range(ek):
                        accs[k] = accs[k] + rows_v[buf, r, pl.ds(k * _LANES, _LANES)]
                return tuple(accs)

            accs = lax.fori_loop(0, sz // _RED_UNROLL, red_body, accs)
            for r in range((sz // _RED_UNROLL) * _RED_UNROLL, sz):
                accs = tuple(
                    accs[k] + rows_v[buf, r, pl.ds(k * _LANES, _LANES)]
                    for k in range(ek)
                )
            return accs

        def finalize(b, accs):
            cnt = jnp.zeros((_LANES,), jnp.int32)
            for j in range(nfull):
                v = x_v[b, pl.ds(j * _LANES, _LANES)]
                cnt = cnt + plsc.all_reduce_population_count(v != 0)
            if tail:
                v = x_v[b, pl.ds(L - _LANES, _LANES)]
                m = (v != 0) & (lax.iota(jnp.int32, _LANES) >= (_LANES - tail))
                cnt = cnt + plsc.all_reduce_population_count(m)
            inv = 1.0 / jnp.maximum(cnt.astype(jnp.float32), 1.0)
            for k in range(ek):
                out_v[b, pl.ds(k * _LANES, _LANES)] = accs[k] * inv

        for j in range(_NBUF - 1):
            start_gather(j // nck, j % nck, j)

        def outer(g, carry):
            b0 = g * rows_per_group
            for u in range(_NBUF):
                s = g * _NBUF + u
                b = b0 + u // nck
                ci = u % nck
                wait_gather(ci, u)
                ns = s + (_NBUF - 1)
                ns_ci = (u + _NBUF - 1) % nck

                @pl.when(ns < bpw * nck)
                def _():
                    start_gather(ns // nck, ns_ci, (u + _NBUF - 1) % _NBUF)

                if ci == 0:
                    accs = (jnp.zeros((_LANES,), jnp.float32),) * ek
                accs = accumulate(u, chunks[ci][1], accs)
                if ci == nck - 1:
                    finalize(b, accs)
            return carry

        lax.fori_loop(0, bpw // rows_per_group, outer, 0)
        pltpu.sync_copy(out_v, out_hbm.at[pl.ds(base, bpw)])

    return enc


def kernel(x, lengths, table):
    del lengths
    B, L = x.shape
    V, E = table.shape
    return _build(B, L, V, E)(x.astype(jnp.int32), table)
